# Initial kernel scaffold; baseline (speedup 1.0000x reference)
import jax
import jax.numpy as jnp
from jax.experimental import pallas as pl


def kernel(features, edge_index, Wl1, Wr1, attn1, bias1, res1, Wl2, Wr2, attn2, bias2, Wl3, Wr3, attn3, bias3, res3):
    raise NotImplementedError("write your pallas kernel here")


# speedup vs baseline: 6.5382x; 6.5382x over previous
import functools

import jax
import jax.numpy as jnp
from jax import lax
from jax.experimental import pallas as pl
from jax.experimental.pallas import tpu as pltpu
from jax.experimental.pallas import tpu_sc as plsc

N = 10000
E = 160000
D = 256
F = 256
NC = 2
NS = 16
NW = NC * NS
EPW = E // NW
CH = 64
PAD_E = E + 4000


def _mesh():
    return plsc.VectorSubcoreMesh(core_axis_name="c", subcore_axis_name="s")


def _extract(ref32, j):
    return ref32[pl.ds((j // 16) * 16, 16)][j % 16]


def _mm(x, w):
    M, K = x.shape
    K2, Nn = w.shape
    BM, BN = 1000, 512
    assert M % BM == 0 and Nn % BN == 0

    def body(x_ref, w_ref, o_ref):
        o_ref[...] = jnp.dot(x_ref[...], w_ref[...],
                             preferred_element_type=jnp.float32)

    return pl.pallas_call(
        body,
        grid=(M // BM, Nn // BN),
        in_specs=[pl.BlockSpec((BM, K), lambda i, j: (i, 0)),
                  pl.BlockSpec((K, BN), lambda i, j: (0, j))],
        out_specs=pl.BlockSpec((BM, BN), lambda i, j: (i, j)),
        out_shape=jax.ShapeDtypeStruct((M, Nn), jnp.float32),
    )(x, w)


def _sc_gather(fs, fd, src_s, dst_s, HF):
    nch = EPW // CH
    last = EPW - CH

    @functools.partial(
        pl.kernel,
        out_type=(jax.ShapeDtypeStruct((E, HF), jnp.float32),
                  jax.ShapeDtypeStruct((E, HF), jnp.float32)),
        mesh=_mesh(),
        scratch_types=[pltpu.VMEM((CH,), jnp.int32),
                       pltpu.VMEM((CH, HF), jnp.float32),
                       pltpu.SemaphoreType.DMA],
    )
    def k(fs_hbm, fd_hbm, si_hbm, di_hbm, ofs_hbm, ofd_hbm, idx_v, rows_v, sem):
        c = lax.axis_index("c")
        s = lax.axis_index("s")
        wid = s * NC + c
        base = wid * EPW

        def chunk(i, carry):
            e0 = base + lax.min(i * CH, last)
            pltpu.sync_copy(si_hbm.at[pl.ds(e0, CH)], idx_v)
            pltpu.async_copy(fs_hbm.at[idx_v], rows_v, sem).wait()
            pltpu.sync_copy(rows_v, ofs_hbm.at[pl.ds(e0, CH)])
            pltpu.sync_copy(di_hbm.at[pl.ds(e0, CH)], idx_v)
            pltpu.async_copy(fd_hbm.at[idx_v], rows_v, sem).wait()
            pltpu.sync_copy(rows_v, ofd_hbm.at[pl.ds(e0, CH)])
            return carry

        nloops = nch + (1 if EPW % CH else 0)
        lax.fori_loop(0, nloops, chunk, 0)

    return k(fs, fd, src_s, dst_s)


def _edge(FS, FD, attn, H):
    HF = H * F
    HFA = HF + 128
    BE = 1000

    def body(fs_ref, fd_ref, attn_ref, o_ref):
        fs = fs_ref[...]
        z = fs + fd_ref[...]
        z = jnp.where(z >= 0, z, 0.2 * z)
        at = attn_ref[...]
        cols = []
        exs = []
        for h in range(H):
            sl = slice(h * F, (h + 1) * F)
            lh = jnp.sum(z[:, sl] * at[h][None, :], axis=1)
            eh = jnp.exp(lh)
            exs.append(eh[:, None])
            cols.append(fs[:, sl] * eh[:, None])
        cols.append(jnp.concatenate(exs, axis=1))
        cols.append(jnp.zeros((BE, 128 - H), jnp.float32))
        o_ref[...] = jnp.concatenate(cols, axis=1)

    return pl.pallas_call(
        body,
        grid=(E // BE,),
        in_specs=[pl.BlockSpec((BE, HF), lambda i: (i, 0)),
                  pl.BlockSpec((BE, HF), lambda i: (i, 0)),
                  pl.BlockSpec((H, F), lambda i: (0, 0))],
        out_specs=pl.BlockSpec((BE, HFA), lambda i: (i, 0)),
        out_shape=jax.ShapeDtypeStruct((PAD_E, HFA), jnp.float32),
    )(FS, FD, attn)


def _sc_scatter(aug, dloc, bnds, zeros_hbm, H, W, NWIN, SCH):
    HF = H * F
    HFA = HF + 128
    OUTR = NWIN * W + 16
    ZPT = W // NS

    @functools.partial(
        pl.kernel,
        out_type=jax.ShapeDtypeStruct((OUTR, HFA), jnp.float32),
        mesh=_mesh(),
        scratch_types=[pltpu.VMEM((32,), jnp.int32),
                       pltpu.VMEM((SCH,), jnp.int32),
                       pltpu.VMEM((SCH,), jnp.int32),
                       pltpu.VMEM((SCH, HFA), jnp.float32),
                       pltpu.VMEM((16, HFA), jnp.float32),
                       pltpu.SemaphoreType.DMA],
    )
    def k(aug_hbm, dloc_hbm, bnds_hbm, z_hbm, out_hbm,
          bnds_v, dvm, lidvm, rows_v, zbuf, sem):
        c = lax.axis_index("c")
        s = lax.axis_index("s")
        wid = s * NC + c
        pltpu.sync_copy(bnds_hbm, bnds_v)
        pltpu.sync_copy(z_hbm, zbuf)
        io16 = lax.iota(jnp.int32, 16)

        for w in range(NWIN):
            wbase = w * W

            @pl.when(c == (w % 2))
            def _window():
                e0w = _extract(bnds_v, w)
                e1w = _extract(bnds_v, w + 1)
                for j in range(ZPT // 16):
                    r0 = wbase + s * ZPT + j * 16
                    pltpu.sync_copy(zbuf, out_hbm.at[pl.ds(r0, 16)])
                plsc.subcore_barrier()
                e0al = (e0w // SCH) * SCH
                span = e1w - e0al
                per = (span + NW - 1) // NW
                per_c = ((per + SCH - 1) // SCH) * SCH
                start_t = e0al + wid * per_c

                def chunk(i, carry):
                    ec = start_t + i * SCH
                    pltpu.sync_copy(dloc_hbm.at[pl.ds(ec, SCH)], dvm)
                    for kk in range(SCH // 16):
                        dv = dvm[pl.ds(kk * 16, 16)]
                        ev = ec + kk * 16 + io16
                        valid = (ev >= e0w) & (ev < e1w)
                        lidvm[pl.ds(kk * 16, 16)] = jnp.where(
                            valid, dv, jnp.int32(NWIN * W))
                    pltpu.sync_copy(aug_hbm.at[pl.ds(ec, SCH)], rows_v)
                    pltpu.sync_copy(rows_v, out_hbm.at[lidvm], add=True)
                    return carry

                lax.fori_loop(0, per_c // SCH, chunk, 0)
                plsc.subcore_barrier()

    return k(aug, dloc, bnds, zeros_hbm)


def _finish(num_aug, res, bias2d, H, act, mean_heads):
    HF = H * F
    HFA = HF + 128
    BN = 1000
    OUTC = F if mean_heads else HF

    def body(a_ref, r_ref, b_ref, o_ref):
        a = a_ref[...]
        r = r_ref[...]
        b = b_ref[...][0]
        acc = None
        cols = []
        for h in range(H):
            sl = slice(h * F, (h + 1) * F)
            den = a[:, HF + h]
            oh = a[:, sl] / (den + 1e-9)[:, None] + r[:, sl] + b[sl][None, :]
            if mean_heads:
                acc = oh if acc is None else acc + oh
            else:
                cols.append(oh)
        if mean_heads:
            o_ref[...] = acc * (1.0 / H)
        else:
            out = jnp.concatenate(cols, axis=1)
            if act:
                out = jnp.maximum(out, 0.0)
            o_ref[...] = out

    return pl.pallas_call(
        body,
        grid=(N // BN,),
        in_specs=[pl.BlockSpec((BN, HFA), lambda i: (i, 0)),
                  pl.BlockSpec((BN, HF), lambda i: (i, 0)),
                  pl.BlockSpec((8, HF), lambda i: (0, 0))],
        out_specs=pl.BlockSpec((BN, OUTC), lambda i: (i, 0)),
        out_shape=jax.ShapeDtypeStruct((N, OUTC), jnp.float32),
    )(num_aug, res, bias2d)


def _layer(x, src_s, dst_s, dloc, zeros16, Wl, Wr, attn, bias, resW,
           H, W, NWIN, bnds, act, mean_heads):
    HF = H * F
    parts = [Wl, Wr] + ([resW] if resW is not None else [])
    proj = _mm(x, jnp.concatenate(parts, axis=1))
    fs = proj[:, :HF]
    fd = proj[:, HF:2 * HF]
    res = proj[:, 2 * HF:] if resW is not None else x
    FS, FD = _sc_gather(fs, fd, src_s, dst_s, HF)
    aug = _edge(FS, FD, attn, H)
    num_aug = _sc_scatter(aug, dloc, bnds, zeros16[:, :HF + 128], H, W, NWIN,
                          64 if H == 4 else 48)
    bias2d = jnp.broadcast_to(bias.reshape(1, HF), (8, HF))
    return _finish(num_aug[:N], res, bias2d, H, act, mean_heads)


def kernel(features, edge_index, Wl1, Wr1, attn1, bias1, res1,
           Wl2, Wr2, attn2, bias2, Wl3, Wr3, attn3, bias3, res3):
    src = edge_index[0]
    dst = edge_index[1]
    order = jnp.argsort(dst)
    src_s = src[order].astype(jnp.int32)
    dst_s = dst[order].astype(jnp.int32)
    dloc = jnp.pad(dst_s, (0, PAD_E - E))
    zeros16 = jnp.zeros((16, 6 * F + 128), jnp.float32)

    W12, NWIN12 = 1024, 10
    W3, NWIN3 = 1024, 10
    b12 = jnp.searchsorted(dst_s, jnp.arange(NWIN12 + 1) * W12).astype(jnp.int32)
    b12 = jnp.pad(b12, (0, 32 - (NWIN12 + 1)))
    b3 = jnp.searchsorted(dst_s, jnp.arange(NWIN3 + 1) * W3).astype(jnp.int32)
    b3 = jnp.pad(b3, (0, 32 - (NWIN3 + 1)))

    h = _layer(features, src_s, dst_s, dloc, zeros16, Wl1, Wr1, attn1, bias1,
               res1, 4, W12, NWIN12, b12, True, False)
    h = _layer(h, src_s, dst_s, dloc, zeros16, Wl2, Wr2, attn2, bias2,
               None, 4, W12, NWIN12, b12, True, False)
    h = _layer(h, src_s, dst_s, dloc, zeros16, Wl3, Wr3, attn3, bias3,
               res3, 6, W3, NWIN3, b3, False, True)
    return h
